# baseline (speedup 1.0000x reference)
import functools

import jax
import jax.numpy as jnp
from jax import lax
from jax.experimental import pallas as pl
from jax.experimental.pallas import tpu as pltpu
from jax.experimental.pallas import tpu_sc as plsc

_N = 10000
_NP = 10240
_D = 128
_E = 320000
_NW = 32
_EW = 10240
_EPAD = _EW * _NW
_CH = 128
_CPW = _EW // _CH
_RPS = _NP // 16
_B = 1024
_GRID = _NP // _B

_f32 = jnp.float32
_i32 = jnp.int32


def _sc_mesh():
    return plsc.VectorSubcoreMesh(core_axis_name="c", subcore_axis_name="s")


_PH = 2
_PC = 80 // _PH


def _seg_body(with_cnt, x_hbm, src_hbm, dstp_hbm, z_hbm, zr_hbm, one_hbm,
              *rest):
    if with_cnt:
        (out_s, out_c, sidx_all, didx_all, rows0, rows1,
         ones_v, cbuf_v, acc_sh, cnt_sh, sem0, sem1) = rest
    else:
        (out_s, sidx_all, didx_all, rows0, rows1,
         acc_sh, sem0, sem1) = rest
    rows = (rows0, rows1)
    sems = (sem0, sem1)
    c = lax.axis_index("c")
    s = lax.axis_index("s")
    wid = c * 16 + s
    r0 = s * _RPS
    pltpu.sync_copy(z_hbm, rows0)
    for t in range(_RPS // _CH):
        pltpu.sync_copy(rows0, acc_sh.at[pl.ds(r0 + t * _CH, _CH)])
    if with_cnt:
        pltpu.sync_copy(zr_hbm, cbuf_v)
        pltpu.sync_copy(cbuf_v, cnt_sh.at[pl.ds(r0, _RPS)])
        pltpu.sync_copy(one_hbm, ones_v)
    plsc.subcore_barrier()

    def fire(t, b):
        pltpu.async_copy(x_hbm.at[sidx_all.at[t]], rows[b], sems[b])

    def commit(t, b):
        pltpu.make_async_copy(x_hbm.at[pl.ds(0, _CH)], rows[b], sems[b]).wait()
        pltpu.sync_copy(rows[b], acc_sh.at[didx_all.at[t]], add=True)
        if with_cnt:
            pltpu.sync_copy(ones_v, cnt_sh.at[didx_all.at[t]], add=True)

    for phase in range(_PH):
        pltpu.sync_copy(src_hbm.at[wid, pl.ds(phase * _PC, _PC)], sidx_all)
        pltpu.sync_copy(dstp_hbm.at[wid, pl.ds(phase * _PC, _PC)], didx_all)
        fire(0, 0)
        fire(1, 1)

        def body(j, carry):
            t = j * 2
            commit(t, 0)
            fire(t + 2, 0)
            commit(t + 1, 1)
            fire(t + 3, 1)
            return carry

        lax.fori_loop(0, _PC // 2 - 1, body, 0)
        commit(_PC - 2, 0)
        commit(_PC - 1, 1)
    plsc.subcore_barrier()
    for t in range(_RPS // _CH):
        pltpu.sync_copy(acc_sh.at[pl.ds(r0 + t * _CH, _CH)], rows0)
        pltpu.sync_copy(rows0, out_s.at[c, pl.ds(r0 + t * _CH, _CH)])
    if with_cnt:
        pltpu.sync_copy(cnt_sh.at[pl.ds(r0, _RPS)], cbuf_v)
        pltpu.sync_copy(cbuf_v, out_c.at[c, pl.ds(r0, _RPS)])


@functools.lru_cache(maxsize=None)
def _seg_kernel(with_cnt):
    if with_cnt:
        out_type = (jax.ShapeDtypeStruct((2, _NP, _D), _f32),
                    jax.ShapeDtypeStruct((2, _NP), _f32))
    else:
        out_type = jax.ShapeDtypeStruct((2, _NP, _D), _f32)
    scratch = [
        pltpu.VMEM((_PC, _CH), _i32),
        pltpu.VMEM((_PC, _CH), _i32),
        pltpu.VMEM((_CH, _D), _f32),
        pltpu.VMEM((_CH, _D), _f32),
    ]
    if with_cnt:
        scratch += [pltpu.VMEM((_CH,), _f32), pltpu.VMEM((_RPS,), _f32)]
    scratch.append(pltpu.VMEM_SHARED((_NP, _D), _f32))
    if with_cnt:
        scratch.append(pltpu.VMEM_SHARED((_NP,), _f32))
    scratch += [pltpu.SemaphoreType.DMA] * 2
    return pl.kernel(
        functools.partial(_seg_body, with_cnt),
        mesh=_sc_mesh(),
        out_type=out_type,
        scratch_types=scratch,
    )


def _seg(xp, srcp, dstp, z128, zrow, one128, with_cnt=True):
    r = _seg_kernel(with_cnt)(xp, srcp.reshape(_NW, _CPW, _CH),
                              dstp.reshape(_NW, _CPW, _CH), z128, zrow, one128)
    return r if with_cnt else (r, None)


def _maskedge_body(mask_hbm, src_hbm, dst_hbm, out_hbm,
                   sv, dv, smv, dmv, dpv, sem, sem2):
    c = lax.axis_index("c")
    s = lax.axis_index("s")
    wid = c * 16 + s
    base = wid * _EW
    dump = jnp.full((16,), _N, _i32)

    def body(t, carry):
        off = pl.multiple_of(base + t * _CH, _CH)
        pltpu.sync_copy(src_hbm.at[pl.ds(off, _CH)], sv)
        pltpu.sync_copy(dst_hbm.at[pl.ds(off, _CH)], dv)
        cp1 = pltpu.async_copy(mask_hbm.at[sv], smv, sem)
        cp2 = pltpu.async_copy(mask_hbm.at[dv], dmv, sem2)
        cp1.wait()
        cp2.wait()
        for i in range(_CH // 16):
            sl = pl.ds(i * 16, 16)
            keep = (smv[sl] > 0) & (dmv[sl] > 0)
            dpv[sl] = jnp.where(keep, dv[sl], dump)
        pltpu.sync_copy(dpv, out_hbm.at[pl.ds(off, _CH)])
        return carry

    lax.fori_loop(0, _CPW, body, 0)


@functools.lru_cache(maxsize=None)
def _maskedge_kernel():
    return pl.kernel(
        _maskedge_body,
        mesh=_sc_mesh(),
        out_type=jax.ShapeDtypeStruct((_EPAD,), _i32),
        scratch_types=[
            pltpu.VMEM((_CH,), _i32),
            pltpu.VMEM((_CH,), _i32),
            pltpu.VMEM((_CH,), _i32),
            pltpu.VMEM((_CH,), _i32),
            pltpu.VMEM((_CH,), _i32),
            pltpu.SemaphoreType.DMA,
            pltpu.SemaphoreType.DMA,
        ],
    )


@functools.lru_cache(maxsize=None)
def _sagelin(ns, nx, relu, use_mask, use_bias=True):
    def body(*refs):
        it = iter(refs)
        cnt = next(it) if ns else None
        ss = [next(it) for _ in range(ns)]
        wls = [next(it) for _ in range(ns)]
        xs = [next(it) for _ in range(nx)]
        wrs = [next(it) for _ in range(nx)]
        bl = next(it) if use_bias else None
        mk = next(it) if use_mask else None
        out = next(it)
        acc = jnp.zeros((_B, _D), _f32)
        if ns:
            cv = jnp.maximum(cnt[0] + cnt[1], 1.0)
            for sref, wl in zip(ss, wls):
                mean = (sref[0] + sref[1]) / cv
                acc = acc + jnp.dot(mean, wl[...], preferred_element_type=_f32)
        for xref, wr in zip(xs, wrs):
            acc = acc + jnp.dot(xref[...], wr[...], preferred_element_type=_f32)
        if use_bias:
            acc = acc + bl[...]
        if relu:
            acc = jnp.maximum(acc, 0.0)
        if use_mask:
            acc = jnp.where(mk[...] > 0, acc, 0.0)
        out[...] = acc

    in_specs = []
    if ns:
        in_specs.append(pl.BlockSpec((2, _B, 1), lambda i: (0, i, 0)))
        in_specs += [pl.BlockSpec((2, _B, _D), lambda i: (0, i, 0))] * ns
        in_specs += [pl.BlockSpec((_D, _D), lambda i: (0, 0))] * ns
    in_specs += [pl.BlockSpec((_B, _D), lambda i: (i, 0))] * nx
    in_specs += [pl.BlockSpec((_D, _D), lambda i: (0, 0))] * nx
    if use_bias:
        in_specs.append(pl.BlockSpec((1, _D), lambda i: (0, 0)))
    if use_mask:
        in_specs.append(pl.BlockSpec((_B, 1), lambda i: (i, 0)))

    call = pl.pallas_call(
        body,
        grid=(_GRID,),
        in_specs=in_specs,
        out_specs=pl.BlockSpec((_B, _D), lambda i: (i, 0)),
        out_shape=jax.ShapeDtypeStruct((_NP, _D), _f32),
        compiler_params=pltpu.CompilerParams(
            dimension_semantics=("arbitrary",)),
    )

    def run(cnt, ss, wls, xs, wrs, bl, mk):
        args = []
        if ns:
            args.append(cnt.reshape(2, _NP, 1))
            args += list(ss)
            args += list(wls)
        args += list(xs)
        args += list(wrs)
        if use_bias:
            args.append(bl.reshape(1, _D))
        if use_mask:
            args.append(mk)
        return call(*args)

    return run


@functools.lru_cache(maxsize=None)
def _pool(k):
    def body(x_ref, m_ref, p_ref, xg_ref, nm_ref):
        xs = x_ref[...]
        pv = p_ref[...]
        pn = jnp.sqrt(jnp.sum(pv * pv)) + 1e-16
        sc = jnp.dot(xs, pv, preferred_element_type=_f32) / pn
        u = lax.bitcast_convert_type(sc, jnp.uint32)
        key = jnp.where(u >= jnp.uint32(0x80000000), ~u,
                        u | jnp.uint32(0x80000000))
        key = jnp.where(m_ref[...] > 0, key, jnp.uint32(0))
        kf = jnp.float32(k)

        def tb(b, t):
            cand = t | (jnp.uint32(1) << (31 - b).astype(jnp.uint32))
            n_ge = jnp.sum((key >= cand).astype(_f32))
            return jnp.where(n_ge >= kf, cand, t)

        T = lax.fori_loop(0, 32, tb, jnp.uint32(0))
        c_gt = jnp.sum((key > T).astype(_f32))
        need = kf - c_gt
        idx = lax.broadcasted_iota(_i32, (_NP, 1), 0)

        def ib(b, m):
            cand = m + (jnp.int32(1) << (13 - b).astype(_i32))
            f = jnp.sum(((key == T) & (idx <= cand - 1)).astype(_f32))
            return jnp.where(f < need, cand, m)

        m = lax.fori_loop(0, 14, ib, jnp.int32(0))
        newm = (key > T) | ((key == T) & (idx <= m))
        nm_ref[...] = newm.astype(_i32)
        gate = jnp.where(newm, jnp.tanh(sc), 0.0)
        xg_ref[...] = xs * gate

    return pl.pallas_call(
        body,
        out_shape=(jax.ShapeDtypeStruct((_NP, _D), _f32),
                   jax.ShapeDtypeStruct((_NP, 1), _i32)),
    )


def kernel(x, edge_index, batch,
           down0_w1l, down0_b1, down0_w1r, down0_w2l, down0_b2, down0_w2r, pool0_p,
           down1_w1l, down1_b1, down1_w1r, down1_w2l, down1_b2, down1_w2r, pool1_p,
           down2_w1l, down2_b1, down2_w1r, down2_w2l, down2_b2, down2_w2r, pool2_p,
           up0_w1l, up0_b1, up0_w1r, up0_w2l, up0_b2, up0_w2r,
           up1_w1l, up1_b1, up1_w1r, up1_w2l, up1_b2, up1_w2r,
           lin1_w, lin1_b):
    x0p = jnp.concatenate(
        [x + batch[:, None].astype(_f32), jnp.zeros((_NP - _N, _D), _f32)], 0)
    src = edge_index[0]
    dst = edge_index[1]
    srcp = jnp.concatenate([src, jnp.zeros((_EPAD - _E,), _i32)])
    dstp0 = jnp.concatenate([dst, jnp.full((_EPAD - _E,), _N, _i32)])
    z128 = jnp.zeros((_CH, _D), _f32)
    zrow = jnp.zeros((_RPS,), _f32)
    one128 = jnp.ones((_CH,), _f32)
    m0col = jnp.concatenate(
        [jnp.ones((_N,), _i32), jnp.zeros((_NP - _N,), _i32)]).reshape(_NP, 1)

    def seg(xp, dstp, want_cnt=False):
        return _seg(xp, srcp, dstp, z128, zrow, one128, with_cnt=want_cnt)

    sA, c0 = seg(x0p, dstp0, True)
    h = _sagelin(1, 1, True, False)(c0, [sA], [down0_w1l], [x0p], [down0_w1r],
                                    down0_b1, None)
    s2, _ = seg(h, dstp0)
    x1pre = _sagelin(1, 1, False, True)(c0, [s2], [down0_w2l], [h], [down0_w2r],
                                        down0_b2, m0col)
    xg1, m1col = _pool(5000)(x1pre, m0col, pool0_p.reshape(_D, 1))
    dstp1 = _maskedge_kernel()(m1col.reshape(_NP), srcp, dstp0)

    sB, c1 = seg(xg1, dstp1, True)
    h = _sagelin(1, 1, True, False)(c1, [sB], [down1_w1l], [xg1], [down1_w1r],
                                    down1_b1, None)
    s2, _ = seg(h, dstp1)
    x2pre = _sagelin(1, 1, False, True)(c1, [s2], [down1_w2l], [h], [down1_w2r],
                                        down1_b2, m1col)
    xg2, m2col = _pool(2500)(x2pre, m1col, pool1_p.reshape(_D, 1))
    dstp2 = _maskedge_kernel()(m2col.reshape(_NP), srcp, dstp0)

    sC, c2 = seg(xg2, dstp2, True)
    h = _sagelin(1, 1, True, False)(c2, [sC], [down2_w1l], [xg2], [down2_w1r],
                                    down2_b1, None)
    s2, _ = seg(h, dstp2)
    x3pre = _sagelin(1, 1, False, True)(c2, [s2], [down2_w2l], [h], [down2_w2r],
                                        down2_b2, m2col)
    xg3, _m3 = _pool(1250)(x3pre, m2col, pool2_p.reshape(_D, 1))

    sX, _ = seg(xg3, dstp1)
    h = _sagelin(2, 2, True, False)(
        c1, [sX, sB], [up1_w1l[:_D], up1_w1l[_D:]],
        [xg3, xg1], [up1_w1r[:_D], up1_w1r[_D:]], up1_b1, None)
    s2, _ = seg(h, dstp1)
    xu = _sagelin(1, 1, False, True)(c1, [s2], [up1_w2l], [h], [up1_w2r],
                                     up1_b2, m1col)

    sY, _ = seg(xu, dstp0)
    h = _sagelin(2, 2, True, False)(
        c0, [sY, sA], [up0_w1l[:_D], up0_w1l[_D:]],
        [xu, x0p], [up0_w1r[:_D], up0_w1r[_D:]], up0_b1, None)
    s2, _ = seg(h, dstp0)
    xf = _sagelin(1, 1, False, True)(c0, [s2], [up0_w2l], [h], [up0_w2r],
                                     up0_b2, m0col)

    out = _sagelin(0, 4, True, False)(
        None, [], [], [x0p, xg1, xg2, xf],
        [lin1_w[0:_D], lin1_w[_D:2 * _D], lin1_w[2 * _D:3 * _D], lin1_w[3 * _D:]],
        lin1_b, None)
    return out[:_N]


# speedup vs baseline: 1.0008x; 1.0008x over previous
import functools

import jax
import jax.numpy as jnp
from jax import lax
from jax.experimental import pallas as pl
from jax.experimental.pallas import tpu as pltpu
from jax.experimental.pallas import tpu_sc as plsc

_N = 10000
_NP = 10240
_D = 128
_E = 320000
_NW = 32
_EW = 10240
_EPAD = _EW * _NW
_CH = 128
_CPW = _EW // _CH
_RPS = _NP // 16
_B = 1024
_GRID = _NP // _B

_f32 = jnp.float32
_i32 = jnp.int32


def _sc_mesh():
    return plsc.VectorSubcoreMesh(core_axis_name="c", subcore_axis_name="s")


_PH = 2
_PC = 80 // _PH


def _seg_body(with_cnt, x_hbm, src_hbm, dstp_hbm, z_hbm, zr_hbm, one_hbm,
              *rest):
    if with_cnt:
        (out_s, out_c, sidx0, sidx1, didx0, didx1, rows0, rows1,
         ones_v, cbuf_v, acc_sh, cnt_sh, sem0, sem1) = rest
    else:
        (out_s, sidx0, sidx1, didx0, didx1, rows0, rows1,
         acc_sh, sem0, sem1) = rest
    sidxs = (sidx0, sidx1)
    didxs = (didx0, didx1)
    rows = (rows0, rows1)
    sems = (sem0, sem1)
    c = lax.axis_index("c")
    s = lax.axis_index("s")
    wid = c * 16 + s
    r0 = s * _RPS
    pltpu.sync_copy(z_hbm, rows0)
    for t in range(_RPS // _CH):
        pltpu.sync_copy(rows0, acc_sh.at[pl.ds(r0 + t * _CH, _CH)])
    if with_cnt:
        pltpu.sync_copy(zr_hbm, cbuf_v)
        pltpu.sync_copy(cbuf_v, cnt_sh.at[pl.ds(r0, _RPS)])
        pltpu.sync_copy(one_hbm, ones_v)
    plsc.subcore_barrier()
    base = wid * _EW

    def fire(t, b):
        off = pl.multiple_of(base + t * _CH, _CH)
        pltpu.sync_copy(src_hbm.at[pl.ds(off, _CH)], sidxs[b])
        pltpu.sync_copy(dstp_hbm.at[pl.ds(off, _CH)], didxs[b])
        pltpu.async_copy(x_hbm.at[sidxs[b]], rows[b], sems[b])

    def commit(b):
        pltpu.make_async_copy(x_hbm.at[pl.ds(0, _CH)], rows[b], sems[b]).wait()
        pltpu.sync_copy(rows[b], acc_sh.at[didxs[b]], add=True)
        if with_cnt:
            pltpu.sync_copy(ones_v, cnt_sh.at[didxs[b]], add=True)

    fire(0, 0)
    fire(1, 1)

    def body(j, carry):
        t = j * 2
        commit(0)
        fire(t + 2, 0)
        commit(1)
        fire(t + 3, 1)
        return carry

    lax.fori_loop(0, _CPW // 2 - 1, body, 0)
    commit(0)
    commit(1)
    plsc.subcore_barrier()
    for t in range(_RPS // _CH):
        pltpu.sync_copy(acc_sh.at[pl.ds(r0 + t * _CH, _CH)], rows0)
        pltpu.sync_copy(rows0, out_s.at[c, pl.ds(r0 + t * _CH, _CH)])
    if with_cnt:
        pltpu.sync_copy(cnt_sh.at[pl.ds(r0, _RPS)], cbuf_v)
        pltpu.sync_copy(cbuf_v, out_c.at[c, pl.ds(r0, _RPS)])


@functools.lru_cache(maxsize=None)
def _seg_kernel(with_cnt):
    if with_cnt:
        out_type = (jax.ShapeDtypeStruct((2, _NP, _D), _f32),
                    jax.ShapeDtypeStruct((2, _NP), _f32))
    else:
        out_type = jax.ShapeDtypeStruct((2, _NP, _D), _f32)
    scratch = [
        pltpu.VMEM((_CH,), _i32),
        pltpu.VMEM((_CH,), _i32),
        pltpu.VMEM((_CH,), _i32),
        pltpu.VMEM((_CH,), _i32),
        pltpu.VMEM((_CH, _D), _f32),
        pltpu.VMEM((_CH, _D), _f32),
    ]
    if with_cnt:
        scratch += [pltpu.VMEM((_CH,), _f32), pltpu.VMEM((_RPS,), _f32)]
    scratch.append(pltpu.VMEM_SHARED((_NP, _D), _f32))
    if with_cnt:
        scratch.append(pltpu.VMEM_SHARED((_NP,), _f32))
    scratch += [pltpu.SemaphoreType.DMA] * 2
    return pl.kernel(
        functools.partial(_seg_body, with_cnt),
        mesh=_sc_mesh(),
        out_type=out_type,
        scratch_types=scratch,
    )


def _seg(xp, srcp, dstp, z128, zrow, one128, with_cnt=True):
    r = _seg_kernel(with_cnt)(xp, srcp, dstp, z128, zrow, one128)
    return r if with_cnt else (r, None)


def _maskedge_body(mask_hbm, src_hbm, dst_hbm, out_hbm,
                   sv, dv, smv, dmv, dpv, sem, sem2):
    c = lax.axis_index("c")
    s = lax.axis_index("s")
    wid = c * 16 + s
    base = wid * _EW
    dump = jnp.full((16,), _N, _i32)

    def body(t, carry):
        off = pl.multiple_of(base + t * _CH, _CH)
        pltpu.sync_copy(src_hbm.at[pl.ds(off, _CH)], sv)
        pltpu.sync_copy(dst_hbm.at[pl.ds(off, _CH)], dv)
        cp1 = pltpu.async_copy(mask_hbm.at[sv], smv, sem)
        cp2 = pltpu.async_copy(mask_hbm.at[dv], dmv, sem2)
        cp1.wait()
        cp2.wait()
        for i in range(_CH // 16):
            sl = pl.ds(i * 16, 16)
            keep = (smv[sl] > 0) & (dmv[sl] > 0)
            dpv[sl] = jnp.where(keep, dv[sl], dump)
        pltpu.sync_copy(dpv, out_hbm.at[pl.ds(off, _CH)])
        return carry

    lax.fori_loop(0, _CPW, body, 0)


@functools.lru_cache(maxsize=None)
def _maskedge_kernel():
    return pl.kernel(
        _maskedge_body,
        mesh=_sc_mesh(),
        out_type=jax.ShapeDtypeStruct((_EPAD,), _i32),
        scratch_types=[
            pltpu.VMEM((_CH,), _i32),
            pltpu.VMEM((_CH,), _i32),
            pltpu.VMEM((_CH,), _i32),
            pltpu.VMEM((_CH,), _i32),
            pltpu.VMEM((_CH,), _i32),
            pltpu.SemaphoreType.DMA,
            pltpu.SemaphoreType.DMA,
        ],
    )


@functools.lru_cache(maxsize=None)
def _sagelin(ns, nx, relu, use_mask, use_bias=True):
    def body(*refs):
        it = iter(refs)
        cnt = next(it) if ns else None
        ss = [next(it) for _ in range(ns)]
        wls = [next(it) for _ in range(ns)]
        xs = [next(it) for _ in range(nx)]
        wrs = [next(it) for _ in range(nx)]
        bl = next(it) if use_bias else None
        mk = next(it) if use_mask else None
        out = next(it)
        acc = jnp.zeros((_B, _D), _f32)
        if ns:
            cv = jnp.maximum(cnt[0] + cnt[1], 1.0)
            for sref, wl in zip(ss, wls):
                mean = (sref[0] + sref[1]) / cv
                acc = acc + jnp.dot(mean, wl[...], preferred_element_type=_f32)
        for xref, wr in zip(xs, wrs):
            acc = acc + jnp.dot(xref[...], wr[...], preferred_element_type=_f32)
        if use_bias:
            acc = acc + bl[...]
        if relu:
            acc = jnp.maximum(acc, 0.0)
        if use_mask:
            acc = jnp.where(mk[...] > 0, acc, 0.0)
        out[...] = acc

    in_specs = []
    if ns:
        in_specs.append(pl.BlockSpec((2, _B, 1), lambda i: (0, i, 0)))
        in_specs += [pl.BlockSpec((2, _B, _D), lambda i: (0, i, 0))] * ns
        in_specs += [pl.BlockSpec((_D, _D), lambda i: (0, 0))] * ns
    in_specs += [pl.BlockSpec((_B, _D), lambda i: (i, 0))] * nx
    in_specs += [pl.BlockSpec((_D, _D), lambda i: (0, 0))] * nx
    if use_bias:
        in_specs.append(pl.BlockSpec((1, _D), lambda i: (0, 0)))
    if use_mask:
        in_specs.append(pl.BlockSpec((_B, 1), lambda i: (i, 0)))

    call = pl.pallas_call(
        body,
        grid=(_GRID,),
        in_specs=in_specs,
        out_specs=pl.BlockSpec((_B, _D), lambda i: (i, 0)),
        out_shape=jax.ShapeDtypeStruct((_NP, _D), _f32),
        compiler_params=pltpu.CompilerParams(
            dimension_semantics=("arbitrary",)),
    )

    def run(cnt, ss, wls, xs, wrs, bl, mk):
        args = []
        if ns:
            args.append(cnt.reshape(2, _NP, 1))
            args += list(ss)
            args += list(wls)
        args += list(xs)
        args += list(wrs)
        if use_bias:
            args.append(bl.reshape(1, _D))
        if use_mask:
            args.append(mk)
        return call(*args)

    return run


@functools.lru_cache(maxsize=None)
def _pool(k):
    def body(x_ref, m_ref, p_ref, xg_ref, nm_ref):
        xs = x_ref[...]
        pv = p_ref[...]
        pn = jnp.sqrt(jnp.sum(pv * pv)) + 1e-16
        sc = jnp.dot(xs, pv, preferred_element_type=_f32) / pn
        u = lax.bitcast_convert_type(sc, jnp.uint32)
        key = jnp.where(u >= jnp.uint32(0x80000000), ~u,
                        u | jnp.uint32(0x80000000))
        key = jnp.where(m_ref[...] > 0, key, jnp.uint32(0))
        kf = jnp.float32(k)

        def tb(b, t):
            cand = t | (jnp.uint32(1) << (31 - b).astype(jnp.uint32))
            n_ge = jnp.sum((key >= cand).astype(_f32))
            return jnp.where(n_ge >= kf, cand, t)

        T = lax.fori_loop(0, 32, tb, jnp.uint32(0))
        c_gt = jnp.sum((key > T).astype(_f32))
        need = kf - c_gt
        idx = lax.broadcasted_iota(_i32, (_NP, 1), 0)

        def ib(b, m):
            cand = m + (jnp.int32(1) << (13 - b).astype(_i32))
            f = jnp.sum(((key == T) & (idx <= cand - 1)).astype(_f32))
            return jnp.where(f < need, cand, m)

        m = lax.fori_loop(0, 14, ib, jnp.int32(0))
        newm = (key > T) | ((key == T) & (idx <= m))
        nm_ref[...] = newm.astype(_i32)
        gate = jnp.where(newm, jnp.tanh(sc), 0.0)
        xg_ref[...] = xs * gate

    return pl.pallas_call(
        body,
        out_shape=(jax.ShapeDtypeStruct((_NP, _D), _f32),
                   jax.ShapeDtypeStruct((_NP, 1), _i32)),
    )


def kernel(x, edge_index, batch,
           down0_w1l, down0_b1, down0_w1r, down0_w2l, down0_b2, down0_w2r, pool0_p,
           down1_w1l, down1_b1, down1_w1r, down1_w2l, down1_b2, down1_w2r, pool1_p,
           down2_w1l, down2_b1, down2_w1r, down2_w2l, down2_b2, down2_w2r, pool2_p,
           up0_w1l, up0_b1, up0_w1r, up0_w2l, up0_b2, up0_w2r,
           up1_w1l, up1_b1, up1_w1r, up1_w2l, up1_b2, up1_w2r,
           lin1_w, lin1_b):
    x0p = jnp.concatenate(
        [x + batch[:, None].astype(_f32), jnp.zeros((_NP - _N, _D), _f32)], 0)
    src = edge_index[0]
    dst = edge_index[1]
    srcp = jnp.concatenate([src, jnp.zeros((_EPAD - _E,), _i32)])
    dstp0 = jnp.concatenate([dst, jnp.full((_EPAD - _E,), _N, _i32)])
    z128 = jnp.zeros((_CH, _D), _f32)
    zrow = jnp.zeros((_RPS,), _f32)
    one128 = jnp.ones((_CH,), _f32)
    m0col = jnp.concatenate(
        [jnp.ones((_N,), _i32), jnp.zeros((_NP - _N,), _i32)]).reshape(_NP, 1)

    def seg(xp, dstp, want_cnt=False):
        return _seg(xp, srcp, dstp, z128, zrow, one128, with_cnt=want_cnt)

    sA, c0 = seg(x0p, dstp0, True)
    h = _sagelin(1, 1, True, False)(c0, [sA], [down0_w1l], [x0p], [down0_w1r],
                                    down0_b1, None)
    s2, _ = seg(h, dstp0)
    x1pre = _sagelin(1, 1, False, True)(c0, [s2], [down0_w2l], [h], [down0_w2r],
                                        down0_b2, m0col)
    xg1, m1col = _pool(5000)(x1pre, m0col, pool0_p.reshape(_D, 1))
    dstp1 = _maskedge_kernel()(m1col.reshape(_NP), srcp, dstp0)

    sB, c1 = seg(xg1, dstp1, True)
    h = _sagelin(1, 1, True, False)(c1, [sB], [down1_w1l], [xg1], [down1_w1r],
                                    down1_b1, None)
    s2, _ = seg(h, dstp1)
    x2pre = _sagelin(1, 1, False, True)(c1, [s2], [down1_w2l], [h], [down1_w2r],
                                        down1_b2, m1col)
    xg2, m2col = _pool(2500)(x2pre, m1col, pool1_p.reshape(_D, 1))
    dstp2 = _maskedge_kernel()(m2col.reshape(_NP), srcp, dstp0)

    sC, c2 = seg(xg2, dstp2, True)
    h = _sagelin(1, 1, True, False)(c2, [sC], [down2_w1l], [xg2], [down2_w1r],
                                    down2_b1, None)
    s2, _ = seg(h, dstp2)
    x3pre = _sagelin(1, 1, False, True)(c2, [s2], [down2_w2l], [h], [down2_w2r],
                                        down2_b2, m2col)
    xg3, _m3 = _pool(1250)(x3pre, m2col, pool2_p.reshape(_D, 1))

    sX, _ = seg(xg3, dstp1)
    h = _sagelin(2, 2, True, False)(
        c1, [sX, sB], [up1_w1l[:_D], up1_w1l[_D:]],
        [xg3, xg1], [up1_w1r[:_D], up1_w1r[_D:]], up1_b1, None)
    s2, _ = seg(h, dstp1)
    xu = _sagelin(1, 1, False, True)(c1, [s2], [up1_w2l], [h], [up1_w2r],
                                     up1_b2, m1col)

    sY, _ = seg(xu, dstp0)
    h = _sagelin(2, 2, True, False)(
        c0, [sY, sA], [up0_w1l[:_D], up0_w1l[_D:]],
        [xu, x0p], [up0_w1r[:_D], up0_w1r[_D:]], up0_b1, None)
    s2, _ = seg(h, dstp0)
    xf = _sagelin(1, 1, False, True)(c0, [s2], [up0_w2l], [h], [up0_w2r],
                                     up0_b2, m0col)

    out = _sagelin(0, 4, True, False)(
        None, [], [], [x0p, xg1, xg2, xf],
        [lin1_w[0:_D], lin1_w[_D:2 * _D], lin1_w[2 * _D:3 * _D], lin1_w[3 * _D:]],
        lin1_b, None)
    return out[:_N]


# speedup vs baseline: 1.0134x; 1.0125x over previous
import functools

import jax
import jax.numpy as jnp
from jax import lax
from jax.experimental import pallas as pl
from jax.experimental.pallas import tpu as pltpu
from jax.experimental.pallas import tpu_sc as plsc

_N = 10000
_NP = 10240
_D = 128
_E = 320000
_NW = 32
_EW = 10240
_EPAD = _EW * _NW
_CH = 128
_CPW = _EW // _CH
_RPS = _NP // 16
_B = 1024
_GRID = _NP // _B

_f32 = jnp.float32
_i32 = jnp.int32


def _sc_mesh():
    return plsc.VectorSubcoreMesh(core_axis_name="c", subcore_axis_name="s")


_PH = 2
_PC = 80 // _PH


def _seg_body(with_cnt, x_hbm, src_hbm, dstp_hbm, z_hbm, zr_hbm, one_hbm,
              *rest):
    if with_cnt:
        (out_s, out_c, sidx0, sidx1, didx0, didx1, rows0, rows1,
         ones_v, cbuf_v, acc_sh, cnt_sh, sem0, sem1) = rest
    else:
        (out_s, sidx0, sidx1, didx0, didx1, rows0, rows1,
         acc_sh, sem0, sem1) = rest
    sidxs = (sidx0, sidx1)
    didxs = (didx0, didx1)
    rows = (rows0, rows1)
    sems = (sem0, sem1)
    c = lax.axis_index("c")
    s = lax.axis_index("s")
    wid = c * 16 + s
    r0 = s * _RPS
    pltpu.sync_copy(z_hbm, rows0)
    for t in range(_RPS // _CH):
        pltpu.sync_copy(rows0, acc_sh.at[pl.ds(r0 + t * _CH, _CH)])
    if with_cnt:
        pltpu.sync_copy(zr_hbm, cbuf_v)
        pltpu.sync_copy(cbuf_v, cnt_sh.at[pl.ds(r0, _RPS)])
        pltpu.sync_copy(one_hbm, ones_v)
    plsc.subcore_barrier()
    base = wid * _EW

    def fire(t, b):
        off = pl.multiple_of(base + t * _CH, _CH)
        pltpu.sync_copy(src_hbm.at[pl.ds(off, _CH)], sidxs[b])
        pltpu.sync_copy(dstp_hbm.at[pl.ds(off, _CH)], didxs[b])
        pltpu.async_copy(x_hbm.at[sidxs[b]], rows[b], sems[b])

    def commit(b):
        pltpu.make_async_copy(x_hbm.at[pl.ds(0, _CH)], rows[b], sems[b]).wait()
        pltpu.sync_copy(rows[b], acc_sh.at[didxs[b]], add=True)
        if with_cnt:
            pltpu.sync_copy(ones_v, cnt_sh.at[didxs[b]], add=True)

    fire(0, 0)
    fire(1, 1)

    def body(j, carry):
        t = j * 2
        commit(0)
        fire(t + 2, 0)
        commit(1)
        fire(t + 3, 1)
        return carry

    lax.fori_loop(0, _CPW // 2 - 1, body, 0)
    commit(0)
    commit(1)
    plsc.subcore_barrier()
    for t in range(_RPS // _CH):
        pltpu.sync_copy(acc_sh.at[pl.ds(r0 + t * _CH, _CH)], rows0)
        pltpu.sync_copy(rows0, out_s.at[c, pl.ds(r0 + t * _CH, _CH)])
    if with_cnt:
        pltpu.sync_copy(cnt_sh.at[pl.ds(r0, _RPS)], cbuf_v)
        pltpu.sync_copy(cbuf_v, out_c.at[c, pl.ds(r0, _RPS)])


@functools.lru_cache(maxsize=None)
def _seg_kernel(with_cnt):
    if with_cnt:
        out_type = (jax.ShapeDtypeStruct((2, _NP, _D), _f32),
                    jax.ShapeDtypeStruct((2, _NP), _f32))
    else:
        out_type = jax.ShapeDtypeStruct((2, _NP, _D), _f32)
    scratch = [
        pltpu.VMEM((_CH,), _i32),
        pltpu.VMEM((_CH,), _i32),
        pltpu.VMEM((_CH,), _i32),
        pltpu.VMEM((_CH,), _i32),
        pltpu.VMEM((_CH, _D), _f32),
        pltpu.VMEM((_CH, _D), _f32),
    ]
    if with_cnt:
        scratch += [pltpu.VMEM((_CH,), _f32), pltpu.VMEM((_RPS,), _f32)]
    scratch.append(pltpu.VMEM_SHARED((_NP, _D), _f32))
    if with_cnt:
        scratch.append(pltpu.VMEM_SHARED((_NP,), _f32))
    scratch += [pltpu.SemaphoreType.DMA] * 2
    return pl.kernel(
        functools.partial(_seg_body, with_cnt),
        mesh=_sc_mesh(),
        out_type=out_type,
        scratch_types=scratch,
    )


def _seg(xp, srcp, dstp, z128, zrow, one128, with_cnt=True):
    r = _seg_kernel(with_cnt)(xp, srcp, dstp, z128, zrow, one128)
    return r if with_cnt else (r, None)


def _maskedge_body(mask_hbm, src_hbm, dst_hbm, out_hbm,
                   sv, dv, smv, dmv, dpv, sem, sem2):
    c = lax.axis_index("c")
    s = lax.axis_index("s")
    wid = c * 16 + s
    base = wid * _EW
    lanes = lax.iota(_i32, 16)

    def body(t, carry):
        off = pl.multiple_of(base + t * _CH, _CH)
        pltpu.sync_copy(src_hbm.at[pl.ds(off, _CH)], sv)
        pltpu.sync_copy(dst_hbm.at[pl.ds(off, _CH)], dv)
        cp1 = pltpu.async_copy(mask_hbm.at[sv], smv, sem)
        cp2 = pltpu.async_copy(mask_hbm.at[dv], dmv, sem2)
        cp1.wait()
        cp2.wait()
        for i in range(_CH // 16):
            sl = pl.ds(i * 16, 16)
            keep = (smv[sl] > 0) & (dmv[sl] > 0)
            dump = _N + ((t * 8 + i) % 15) * 16 + lanes
            dpv[sl] = jnp.where(keep, dv[sl], dump)
        pltpu.sync_copy(dpv, out_hbm.at[pl.ds(off, _CH)])
        return carry

    lax.fori_loop(0, _CPW, body, 0)


@functools.lru_cache(maxsize=None)
def _maskedge_kernel():
    return pl.kernel(
        _maskedge_body,
        mesh=_sc_mesh(),
        out_type=jax.ShapeDtypeStruct((_EPAD,), _i32),
        scratch_types=[
            pltpu.VMEM((_CH,), _i32),
            pltpu.VMEM((_CH,), _i32),
            pltpu.VMEM((_CH,), _i32),
            pltpu.VMEM((_CH,), _i32),
            pltpu.VMEM((_CH,), _i32),
            pltpu.SemaphoreType.DMA,
            pltpu.SemaphoreType.DMA,
        ],
    )


@functools.lru_cache(maxsize=None)
def _sagelin(ns, nx, relu, use_mask, use_bias=True):
    def body(*refs):
        it = iter(refs)
        cnt = next(it) if ns else None
        ss = [next(it) for _ in range(ns)]
        wls = [next(it) for _ in range(ns)]
        xs = [next(it) for _ in range(nx)]
        wrs = [next(it) for _ in range(nx)]
        bl = next(it) if use_bias else None
        mk = next(it) if use_mask else None
        out = next(it)
        acc = jnp.zeros((_B, _D), _f32)
        if ns:
            cv = jnp.maximum(cnt[0] + cnt[1], 1.0)
            for sref, wl in zip(ss, wls):
                mean = (sref[0] + sref[1]) / cv
                acc = acc + jnp.dot(mean, wl[...], preferred_element_type=_f32)
        for xref, wr in zip(xs, wrs):
            acc = acc + jnp.dot(xref[...], wr[...], preferred_element_type=_f32)
        if use_bias:
            acc = acc + bl[...]
        if relu:
            acc = jnp.maximum(acc, 0.0)
        if use_mask:
            acc = jnp.where(mk[...] > 0, acc, 0.0)
        out[...] = acc

    in_specs = []
    if ns:
        in_specs.append(pl.BlockSpec((2, _B, 1), lambda i: (0, i, 0)))
        in_specs += [pl.BlockSpec((2, _B, _D), lambda i: (0, i, 0))] * ns
        in_specs += [pl.BlockSpec((_D, _D), lambda i: (0, 0))] * ns
    in_specs += [pl.BlockSpec((_B, _D), lambda i: (i, 0))] * nx
    in_specs += [pl.BlockSpec((_D, _D), lambda i: (0, 0))] * nx
    if use_bias:
        in_specs.append(pl.BlockSpec((1, _D), lambda i: (0, 0)))
    if use_mask:
        in_specs.append(pl.BlockSpec((_B, 1), lambda i: (i, 0)))

    call = pl.pallas_call(
        body,
        grid=(_GRID,),
        in_specs=in_specs,
        out_specs=pl.BlockSpec((_B, _D), lambda i: (i, 0)),
        out_shape=jax.ShapeDtypeStruct((_NP, _D), _f32),
        compiler_params=pltpu.CompilerParams(
            dimension_semantics=("arbitrary",)),
    )

    def run(cnt, ss, wls, xs, wrs, bl, mk):
        args = []
        if ns:
            args.append(cnt.reshape(2, _NP, 1))
            args += list(ss)
            args += list(wls)
        args += list(xs)
        args += list(wrs)
        if use_bias:
            args.append(bl.reshape(1, _D))
        if use_mask:
            args.append(mk)
        return call(*args)

    return run


@functools.lru_cache(maxsize=None)
def _pool(k):
    def body(x_ref, m_ref, p_ref, xg_ref, nm_ref):
        xs = x_ref[...]
        pv = p_ref[...]
        pn = jnp.sqrt(jnp.sum(pv * pv)) + 1e-16
        sc = jnp.dot(xs, pv, preferred_element_type=_f32) / pn
        u = lax.bitcast_convert_type(sc, jnp.uint32)
        key = jnp.where(u >= jnp.uint32(0x80000000), ~u,
                        u | jnp.uint32(0x80000000))
        key = jnp.where(m_ref[...] > 0, key, jnp.uint32(0))
        kf = jnp.float32(k)

        def tb(b, t):
            cand = t | (jnp.uint32(1) << (31 - b).astype(jnp.uint32))
            n_ge = jnp.sum((key >= cand).astype(_f32))
            return jnp.where(n_ge >= kf, cand, t)

        T = lax.fori_loop(0, 32, tb, jnp.uint32(0))
        c_gt = jnp.sum((key > T).astype(_f32))
        need = kf - c_gt
        idx = lax.broadcasted_iota(_i32, (_NP, 1), 0)

        def ib(b, m):
            cand = m + (jnp.int32(1) << (13 - b).astype(_i32))
            f = jnp.sum(((key == T) & (idx <= cand - 1)).astype(_f32))
            return jnp.where(f < need, cand, m)

        m = lax.fori_loop(0, 14, ib, jnp.int32(0))
        newm = (key > T) | ((key == T) & (idx <= m))
        nm_ref[...] = newm.astype(_i32)
        gate = jnp.where(newm, jnp.tanh(sc), 0.0)
        xg_ref[...] = xs * gate

    return pl.pallas_call(
        body,
        out_shape=(jax.ShapeDtypeStruct((_NP, _D), _f32),
                   jax.ShapeDtypeStruct((_NP, 1), _i32)),
    )


def kernel(x, edge_index, batch,
           down0_w1l, down0_b1, down0_w1r, down0_w2l, down0_b2, down0_w2r, pool0_p,
           down1_w1l, down1_b1, down1_w1r, down1_w2l, down1_b2, down1_w2r, pool1_p,
           down2_w1l, down2_b1, down2_w1r, down2_w2l, down2_b2, down2_w2r, pool2_p,
           up0_w1l, up0_b1, up0_w1r, up0_w2l, up0_b2, up0_w2r,
           up1_w1l, up1_b1, up1_w1r, up1_w2l, up1_b2, up1_w2r,
           lin1_w, lin1_b):
    x0p = jnp.concatenate(
        [x + batch[:, None].astype(_f32), jnp.zeros((_NP - _N, _D), _f32)], 0)
    src = edge_index[0]
    dst = edge_index[1]
    srcp = jnp.concatenate([src, jnp.zeros((_EPAD - _E,), _i32)])
    dstp0 = jnp.concatenate(
        [dst, _N + (jnp.arange(_EPAD - _E, dtype=_i32) % (_NP - _N))])
    z128 = jnp.zeros((_CH, _D), _f32)
    zrow = jnp.zeros((_RPS,), _f32)
    one128 = jnp.ones((_CH,), _f32)
    m0col = jnp.concatenate(
        [jnp.ones((_N,), _i32), jnp.zeros((_NP - _N,), _i32)]).reshape(_NP, 1)

    def seg(xp, dstp, want_cnt=False):
        return _seg(xp, srcp, dstp, z128, zrow, one128, with_cnt=want_cnt)

    sA, c0 = seg(x0p, dstp0, True)
    h = _sagelin(1, 1, True, False)(c0, [sA], [down0_w1l], [x0p], [down0_w1r],
                                    down0_b1, None)
    s2, _ = seg(h, dstp0)
    x1pre = _sagelin(1, 1, False, True)(c0, [s2], [down0_w2l], [h], [down0_w2r],
                                        down0_b2, m0col)
    xg1, m1col = _pool(5000)(x1pre, m0col, pool0_p.reshape(_D, 1))
    dstp1 = _maskedge_kernel()(m1col.reshape(_NP), srcp, dstp0)

    sB, c1 = seg(xg1, dstp1, True)
    h = _sagelin(1, 1, True, False)(c1, [sB], [down1_w1l], [xg1], [down1_w1r],
                                    down1_b1, None)
    s2, _ = seg(h, dstp1)
    x2pre = _sagelin(1, 1, False, True)(c1, [s2], [down1_w2l], [h], [down1_w2r],
                                        down1_b2, m1col)
    xg2, m2col = _pool(2500)(x2pre, m1col, pool1_p.reshape(_D, 1))
    dstp2 = _maskedge_kernel()(m2col.reshape(_NP), srcp, dstp0)

    sC, c2 = seg(xg2, dstp2, True)
    h = _sagelin(1, 1, True, False)(c2, [sC], [down2_w1l], [xg2], [down2_w1r],
                                    down2_b1, None)
    s2, _ = seg(h, dstp2)
    x3pre = _sagelin(1, 1, False, True)(c2, [s2], [down2_w2l], [h], [down2_w2r],
                                        down2_b2, m2col)
    xg3, _m3 = _pool(1250)(x3pre, m2col, pool2_p.reshape(_D, 1))

    sX, _ = seg(xg3, dstp1)
    h = _sagelin(2, 2, True, False)(
        c1, [sX, sB], [up1_w1l[:_D], up1_w1l[_D:]],
        [xg3, xg1], [up1_w1r[:_D], up1_w1r[_D:]], up1_b1, None)
    s2, _ = seg(h, dstp1)
    xu = _sagelin(1, 1, False, True)(c1, [s2], [up1_w2l], [h], [up1_w2r],
                                     up1_b2, m1col)

    sY, _ = seg(xu, dstp0)
    h = _sagelin(2, 2, True, False)(
        c0, [sY, sA], [up0_w1l[:_D], up0_w1l[_D:]],
        [xu, x0p], [up0_w1r[:_D], up0_w1r[_D:]], up0_b1, None)
    s2, _ = seg(h, dstp0)
    xf = _sagelin(1, 1, False, True)(c0, [s2], [up0_w2l], [h], [up0_w2r],
                                     up0_b2, m0col)

    out = _sagelin(0, 4, True, False)(
        None, [], [], [x0p, xg1, xg2, xf],
        [lin1_w[0:_D], lin1_w[_D:2 * _D], lin1_w[2 * _D:3 * _D], lin1_w[3 * _D:]],
        lin1_b, None)
    return out[:_N]


# speedup vs baseline: 1.0664x; 1.0523x over previous
import functools

import jax
import jax.numpy as jnp
from jax import lax
from jax.experimental import pallas as pl
from jax.experimental.pallas import tpu as pltpu
from jax.experimental.pallas import tpu_sc as plsc

_N = 10000
_NP = 10240
_D = 128
_E = 320000
_NW = 32
_EW = 10240
_EPAD = _EW * _NW
_CH = 128
_CPW = _EW // _CH
_RPS = _NP // 16
_B = 1024
_GRID = _NP // _B

_f32 = jnp.float32
_i32 = jnp.int32


def _sc_mesh():
    return plsc.VectorSubcoreMesh(core_axis_name="c", subcore_axis_name="s")


_PH = 2
_PC = 80 // _PH


def _seg_body(with_cnt, x_hbm, src_hbm, dstp_hbm, z_hbm, zr_hbm, one_hbm,
              *rest):
    if with_cnt:
        (out_s, out_c, sidx0, sidx1, didx0, didx1, rows0, rows1,
         ones_v, cbuf_v, acc_sh, cnt_sh, sem0, sem1) = rest
    else:
        (out_s, sidx0, sidx1, didx0, didx1, rows0, rows1,
         acc_sh, sem0, sem1) = rest
    sidxs = (sidx0, sidx1)
    didxs = (didx0, didx1)
    rows = (rows0, rows1)
    sems = (sem0, sem1)
    c = lax.axis_index("c")
    s = lax.axis_index("s")
    wid = c * 16 + s
    r0 = s * _RPS
    pltpu.sync_copy(z_hbm, rows0)
    for t in range(_RPS // _CH):
        pltpu.sync_copy(rows0, acc_sh.at[pl.ds(r0 + t * _CH, _CH)])
    if with_cnt:
        pltpu.sync_copy(zr_hbm, cbuf_v)
        pltpu.sync_copy(cbuf_v, cnt_sh.at[pl.ds(r0, _RPS)])
        pltpu.sync_copy(one_hbm, ones_v)
    plsc.subcore_barrier()
    base = wid * _EW

    def fire(t, b):
        off = pl.multiple_of(base + t * _CH, _CH)
        pltpu.sync_copy(src_hbm.at[pl.ds(off, _CH)], sidxs[b])
        pltpu.sync_copy(dstp_hbm.at[pl.ds(off, _CH)], didxs[b])
        pltpu.async_copy(x_hbm.at[sidxs[b]], rows[b], sems[b])

    def commit(b):
        pltpu.make_async_copy(x_hbm.at[pl.ds(0, _CH)], rows[b], sems[b]).wait()
        pltpu.sync_copy(rows[b], acc_sh.at[didxs[b]], add=True)
        if with_cnt:
            pltpu.sync_copy(ones_v, cnt_sh.at[didxs[b]], add=True)

    fire(0, 0)
    fire(1, 1)

    def body(j, carry):
        t = j * 2
        commit(0)
        fire(t + 2, 0)
        commit(1)
        fire(t + 3, 1)
        return carry

    lax.fori_loop(0, _CPW // 2 - 1, body, 0)
    commit(0)
    commit(1)
    plsc.subcore_barrier()
    for t in range(_RPS // _CH):
        pltpu.sync_copy(acc_sh.at[pl.ds(r0 + t * _CH, _CH)], rows0)
        pltpu.sync_copy(rows0, out_s.at[c, pl.ds(r0 + t * _CH, _CH)])
    if with_cnt:
        pltpu.sync_copy(cnt_sh.at[pl.ds(r0, _RPS)], cbuf_v)
        pltpu.sync_copy(cbuf_v, out_c.at[c, pl.ds(r0, _RPS)])


@functools.lru_cache(maxsize=None)
def _seg_kernel(with_cnt):
    if with_cnt:
        out_type = (jax.ShapeDtypeStruct((2, _NP, _D), _f32),
                    jax.ShapeDtypeStruct((2, _NP), _f32))
    else:
        out_type = jax.ShapeDtypeStruct((2, _NP, _D), _f32)
    scratch = [
        pltpu.VMEM((_CH,), _i32),
        pltpu.VMEM((_CH,), _i32),
        pltpu.VMEM((_CH,), _i32),
        pltpu.VMEM((_CH,), _i32),
        pltpu.VMEM((_CH, _D), _f32),
        pltpu.VMEM((_CH, _D), _f32),
    ]
    if with_cnt:
        scratch += [pltpu.VMEM((_CH,), _f32), pltpu.VMEM((_RPS,), _f32)]
    scratch.append(pltpu.VMEM_SHARED((_NP, _D), _f32))
    if with_cnt:
        scratch.append(pltpu.VMEM_SHARED((_NP,), _f32))
    scratch += [pltpu.SemaphoreType.DMA] * 2
    return pl.kernel(
        functools.partial(_seg_body, with_cnt),
        mesh=_sc_mesh(),
        out_type=out_type,
        scratch_types=scratch,
    )


def _seg(xp, srcp, dstp, z128, zrow, one128, with_cnt=True):
    r = _seg_kernel(with_cnt)(xp, srcp, dstp, z128, zrow, one128)
    return r if with_cnt else (r, None)


def _maskedge_body(mask_hbm, src_hbm, dst_hbm, out_hbm,
                   sv, dv, smv, dmv, dpv, sem, sem2):
    c = lax.axis_index("c")
    s = lax.axis_index("s")
    wid = c * 16 + s
    base = wid * _EW
    lanes = lax.iota(_i32, 16)

    def body(t, carry):
        off = pl.multiple_of(base + t * _CH, _CH)
        pltpu.sync_copy(src_hbm.at[pl.ds(off, _CH)], sv)
        pltpu.sync_copy(dst_hbm.at[pl.ds(off, _CH)], dv)
        cp1 = pltpu.async_copy(mask_hbm.at[sv], smv, sem)
        cp2 = pltpu.async_copy(mask_hbm.at[dv], dmv, sem2)
        cp1.wait()
        cp2.wait()
        for i in range(_CH // 16):
            sl = pl.ds(i * 16, 16)
            keep = (smv[sl] > 0) & (dmv[sl] > 0)
            dump = _N + ((t * 8 + i) % 15) * 16 + lanes
            dpv[sl] = jnp.where(keep, dv[sl], dump)
        pltpu.sync_copy(dpv, out_hbm.at[pl.ds(off, _CH)])
        return carry

    lax.fori_loop(0, _CPW, body, 0)


@functools.lru_cache(maxsize=None)
def _maskedge_kernel():
    return pl.kernel(
        _maskedge_body,
        mesh=_sc_mesh(),
        out_type=jax.ShapeDtypeStruct((_EPAD,), _i32),
        scratch_types=[
            pltpu.VMEM((_CH,), _i32),
            pltpu.VMEM((_CH,), _i32),
            pltpu.VMEM((_CH,), _i32),
            pltpu.VMEM((_CH,), _i32),
            pltpu.VMEM((_CH,), _i32),
            pltpu.SemaphoreType.DMA,
            pltpu.SemaphoreType.DMA,
        ],
    )


@functools.lru_cache(maxsize=None)
def _sagelin(ns, nx, relu, use_mask, use_bias=True):
    def body(*refs):
        it = iter(refs)
        cnt = next(it) if ns else None
        ss = [next(it) for _ in range(ns)]
        wls = [next(it) for _ in range(ns)]
        xs = [next(it) for _ in range(nx)]
        wrs = [next(it) for _ in range(nx)]
        bl = next(it) if use_bias else None
        mk = next(it) if use_mask else None
        out = next(it)
        acc = jnp.zeros((_B, _D), _f32)
        if ns:
            cv = jnp.maximum(cnt[0] + cnt[1], 1.0)
            for sref, wl in zip(ss, wls):
                mean = (sref[0] + sref[1]) / cv
                acc = acc + jnp.dot(mean, wl[...], preferred_element_type=_f32)
        for xref, wr in zip(xs, wrs):
            acc = acc + jnp.dot(xref[...], wr[...], preferred_element_type=_f32)
        if use_bias:
            acc = acc + bl[...]
        if relu:
            acc = jnp.maximum(acc, 0.0)
        if use_mask:
            acc = jnp.where(mk[...] > 0, acc, 0.0)
        out[...] = acc

    in_specs = []
    if ns:
        in_specs.append(pl.BlockSpec((2, _B, 1), lambda i: (0, i, 0)))
        in_specs += [pl.BlockSpec((2, _B, _D), lambda i: (0, i, 0))] * ns
        in_specs += [pl.BlockSpec((_D, _D), lambda i: (0, 0))] * ns
    in_specs += [pl.BlockSpec((_B, _D), lambda i: (i, 0))] * nx
    in_specs += [pl.BlockSpec((_D, _D), lambda i: (0, 0))] * nx
    if use_bias:
        in_specs.append(pl.BlockSpec((1, _D), lambda i: (0, 0)))
    if use_mask:
        in_specs.append(pl.BlockSpec((_B, 1), lambda i: (i, 0)))

    call = pl.pallas_call(
        body,
        grid=(_GRID,),
        in_specs=in_specs,
        out_specs=pl.BlockSpec((_B, _D), lambda i: (i, 0)),
        out_shape=jax.ShapeDtypeStruct((_NP, _D), _f32),
        compiler_params=pltpu.CompilerParams(
            dimension_semantics=("arbitrary",)),
    )

    def run(cnt, ss, wls, xs, wrs, bl, mk):
        args = []
        if ns:
            args.append(cnt.reshape(2, _NP, 1))
            args += list(ss)
            args += list(wls)
        args += list(xs)
        args += list(wrs)
        if use_bias:
            args.append(bl.reshape(1, _D))
        if use_mask:
            args.append(mk)
        return call(*args)

    return run


@functools.lru_cache(maxsize=None)
def _pool(k):
    def body(x_ref, m_ref, p_ref, xg_ref, nm_ref):
        xs = x_ref[...]
        pv = p_ref[...]
        pn = jnp.sqrt(jnp.sum(pv * pv)) + 1e-16
        sc = jnp.dot(xs, pv, preferred_element_type=_f32) / pn
        u = lax.bitcast_convert_type(sc, jnp.uint32)
        key = jnp.where(u >= jnp.uint32(0x80000000), ~u,
                        u | jnp.uint32(0x80000000))
        key = jnp.where(m_ref[...] > 0, key, jnp.uint32(0))
        kf = jnp.float32(k)

        def tb(b, t):
            cand = t | (jnp.uint32(1) << (31 - b).astype(jnp.uint32))
            n_ge = jnp.sum((key >= cand).astype(_f32))
            return jnp.where(n_ge >= kf, cand, t)

        T = lax.fori_loop(0, 32, tb, jnp.uint32(0))
        c_gt = jnp.sum((key > T).astype(_f32))
        need = kf - c_gt
        idx = lax.broadcasted_iota(_i32, (_NP, 1), 0)

        def ib(b, m):
            cand = m + (jnp.int32(1) << (13 - b).astype(_i32))
            f = jnp.sum(((key == T) & (idx <= cand - 1)).astype(_f32))
            return jnp.where(f < need, cand, m)

        m = lax.fori_loop(0, 14, ib, jnp.int32(0))
        newm = (key > T) | ((key == T) & (idx <= m))
        nm_ref[...] = newm.astype(_i32)
        gate = jnp.where(newm, jnp.tanh(sc), 0.0)
        xg_ref[...] = xs * gate

    return pl.pallas_call(
        body,
        out_shape=(jax.ShapeDtypeStruct((_NP, _D), _f32),
                   jax.ShapeDtypeStruct((_NP, 1), _i32)),
    )


def kernel(x, edge_index, batch,
           down0_w1l, down0_b1, down0_w1r, down0_w2l, down0_b2, down0_w2r, pool0_p,
           down1_w1l, down1_b1, down1_w1r, down1_w2l, down1_b2, down1_w2r, pool1_p,
           down2_w1l, down2_b1, down2_w1r, down2_w2l, down2_b2, down2_w2r, pool2_p,
           up0_w1l, up0_b1, up0_w1r, up0_w2l, up0_b2, up0_w2r,
           up1_w1l, up1_b1, up1_w1r, up1_w2l, up1_b2, up1_w2r,
           lin1_w, lin1_b):
    x0p = jnp.concatenate(
        [x + batch[:, None].astype(_f32), jnp.zeros((_NP - _N, _D), _f32)], 0)
    src = edge_index[0]
    dst = edge_index[1]
    srcp = jnp.concatenate([src, jnp.zeros((_EPAD - _E,), _i32)])
    dstp0 = jnp.concatenate(
        [dst, _N + (jnp.arange(_EPAD - _E, dtype=_i32) % (_NP - _N))])
    z128 = jnp.zeros((_CH, _D), _f32)
    zrow = jnp.zeros((_RPS,), _f32)
    one128 = jnp.ones((_CH,), _f32)
    m0col = jnp.concatenate(
        [jnp.ones((_N,), _i32), jnp.zeros((_NP - _N,), _i32)]).reshape(_NP, 1)

    def seg(xp, dstp, want_cnt=True):
        return _seg(xp, srcp, dstp, z128, zrow, one128, with_cnt=want_cnt)

    sA, c0 = seg(x0p, dstp0, True)
    h = _sagelin(1, 1, True, False)(c0, [sA], [down0_w1l], [x0p], [down0_w1r],
                                    down0_b1, None)
    s2, _ = seg(h, dstp0)
    x1pre = _sagelin(1, 1, False, True)(c0, [s2], [down0_w2l], [h], [down0_w2r],
                                        down0_b2, m0col)
    xg1, m1col = _pool(5000)(x1pre, m0col, pool0_p.reshape(_D, 1))
    dstp1 = _maskedge_kernel()(m1col.reshape(_NP), srcp, dstp0)

    sB, c1 = seg(xg1, dstp1, True)
    h = _sagelin(1, 1, True, False)(c1, [sB], [down1_w1l], [xg1], [down1_w1r],
                                    down1_b1, None)
    s2, _ = seg(h, dstp1)
    x2pre = _sagelin(1, 1, False, True)(c1, [s2], [down1_w2l], [h], [down1_w2r],
                                        down1_b2, m1col)
    xg2, m2col = _pool(2500)(x2pre, m1col, pool1_p.reshape(_D, 1))
    dstp2 = _maskedge_kernel()(m2col.reshape(_NP), srcp, dstp0)

    sC, c2 = seg(xg2, dstp2, True)
    h = _sagelin(1, 1, True, False)(c2, [sC], [down2_w1l], [xg2], [down2_w1r],
                                    down2_b1, None)
    s2, _ = seg(h, dstp2)
    x3pre = _sagelin(1, 1, False, True)(c2, [s2], [down2_w2l], [h], [down2_w2r],
                                        down2_b2, m2col)
    xg3, _m3 = _pool(1250)(x3pre, m2col, pool2_p.reshape(_D, 1))

    sX, _ = seg(xg3, dstp1)
    h = _sagelin(2, 2, True, False)(
        c1, [sX, sB], [up1_w1l[:_D], up1_w1l[_D:]],
        [xg3, xg1], [up1_w1r[:_D], up1_w1r[_D:]], up1_b1, None)
    s2, _ = seg(h, dstp1)
    xu = _sagelin(1, 1, False, True)(c1, [s2], [up1_w2l], [h], [up1_w2r],
                                     up1_b2, m1col)

    sY, _ = seg(xu, dstp0)
    h = _sagelin(2, 2, True, False)(
        c0, [sY, sA], [up0_w1l[:_D], up0_w1l[_D:]],
        [xu, x0p], [up0_w1r[:_D], up0_w1r[_D:]], up0_b1, None)
    s2, _ = seg(h, dstp0)
    xf = _sagelin(1, 1, False, True)(c0, [s2], [up0_w2l], [h], [up0_w2r],
                                     up0_b2, m0col)

    out = _sagelin(0, 4, True, False)(
        None, [], [], [x0p, xg1, xg2, xf],
        [lin1_w[0:_D], lin1_w[_D:2 * _D], lin1_w[2 * _D:3 * _D], lin1_w[3 * _D:]],
        lin1_b, None)
    return out[:_N]


# speedup vs baseline: 2.4346x; 2.2830x over previous
import functools

import jax
import jax.numpy as jnp
from jax import lax
from jax.experimental import pallas as pl
from jax.experimental.pallas import tpu as pltpu
from jax.experimental.pallas import tpu_sc as plsc

_N = 10000
_NP = 10240
_D = 128
_E = 320000
_NW = 32
_EW = 10240
_EPAD = _EW * _NW
_CH = 128
_CPW = _EW // _CH
_RPS = _NP // 16
_B = 1024
_GRID = _NP // _B

_f32 = jnp.float32
_i32 = jnp.int32


def _sc_mesh():
    return plsc.VectorSubcoreMesh(core_axis_name="c", subcore_axis_name="s")


_PH = 2
_PC = 80 // _PH


def _seg_body(with_cnt, x_hbm, src_hbm, dstp_hbm, z_hbm, zr_hbm, one_hbm,
              *rest):
    if with_cnt:
        (out_s, out_c, sidx0, sidx1, didx0, didx1, rows0, rows1,
         ones_v, cbuf_v, acc_sh, cnt_sh, sem0, sem1) = rest
    else:
        (out_s, sidx0, sidx1, didx0, didx1, rows0, rows1,
         acc_sh, sem0, sem1) = rest
    sidxs = (sidx0, sidx1)
    didxs = (didx0, didx1)
    rows = (rows0, rows1)
    sems = (sem0, sem1)
    c = lax.axis_index("c")
    s = lax.axis_index("s")
    wid = c * 16 + s
    r0 = s * _RPS
    pltpu.sync_copy(z_hbm, rows0)
    for t in range(_RPS // _CH):
        pltpu.sync_copy(rows0, acc_sh.at[pl.ds(r0 + t * _CH, _CH)])
    if with_cnt:
        pltpu.sync_copy(zr_hbm, cbuf_v)
        pltpu.sync_copy(cbuf_v, cnt_sh.at[pl.ds(r0, _RPS)])
        pltpu.sync_copy(one_hbm, ones_v)
    plsc.subcore_barrier()
    base = wid * _EW

    def fire(t, b):
        off = pl.multiple_of(base + t * _CH, _CH)
        pltpu.sync_copy(src_hbm.at[pl.ds(off, _CH)], sidxs[b])
        pltpu.sync_copy(dstp_hbm.at[pl.ds(off, _CH)], didxs[b])
        pltpu.async_copy(x_hbm.at[sidxs[b]], rows[b], sems[b])

    def commit(b):
        pltpu.make_async_copy(x_hbm.at[pl.ds(0, _CH)], rows[b], sems[b]).wait()
        pltpu.sync_copy(rows[b], acc_sh.at[didxs[b]], add=True)
        if with_cnt:
            pltpu.sync_copy(ones_v, cnt_sh.at[didxs[b]], add=True)

    fire(0, 0)
    fire(1, 1)

    def body(j, carry):
        t = j * 2
        commit(0)
        fire(t + 2, 0)
        commit(1)
        fire(t + 3, 1)
        return carry

    lax.fori_loop(0, _CPW // 2 - 1, body, 0)
    commit(0)
    commit(1)
    plsc.subcore_barrier()
    for t in range(_RPS // _CH):
        pltpu.sync_copy(acc_sh.at[pl.ds(r0 + t * _CH, _CH)], rows0)
        pltpu.sync_copy(rows0, out_s.at[c, pl.ds(r0 + t * _CH, _CH)])
    if with_cnt:
        pltpu.sync_copy(cnt_sh.at[pl.ds(r0, _RPS)], cbuf_v)
        pltpu.sync_copy(cbuf_v, out_c.at[c, pl.ds(r0, _RPS)])


@functools.lru_cache(maxsize=None)
def _seg_kernel(with_cnt):
    if with_cnt:
        out_type = (jax.ShapeDtypeStruct((2, _NP, _D), _f32),
                    jax.ShapeDtypeStruct((2, _NP), _f32))
    else:
        out_type = jax.ShapeDtypeStruct((2, _NP, _D), _f32)
    scratch = [
        pltpu.VMEM((_CH,), _i32),
        pltpu.VMEM((_CH,), _i32),
        pltpu.VMEM((_CH,), _i32),
        pltpu.VMEM((_CH,), _i32),
        pltpu.VMEM((_CH, _D), _f32),
        pltpu.VMEM((_CH, _D), _f32),
    ]
    if with_cnt:
        scratch += [pltpu.VMEM((_CH,), _f32), pltpu.VMEM((_RPS,), _f32)]
    scratch.append(pltpu.VMEM_SHARED((_NP, _D), _f32))
    if with_cnt:
        scratch.append(pltpu.VMEM_SHARED((_NP,), _f32))
    scratch += [pltpu.SemaphoreType.DMA] * 2
    return pl.kernel(
        functools.partial(_seg_body, with_cnt),
        mesh=_sc_mesh(),
        out_type=out_type,
        scratch_types=scratch,
    )


def _seg(xp, srcp, dstp, z128, zrow, one128, with_cnt=True):
    r = _seg_kernel(with_cnt)(xp, srcp, dstp, z128, zrow, one128)
    return r if with_cnt else (r, None)


def _maskedge_body(mask_hbm, src_hbm, dst_hbm, out_hbm,
                   sv, dv, smv, dmv, dpv, sem, sem2):
    c = lax.axis_index("c")
    s = lax.axis_index("s")
    wid = c * 16 + s
    base = wid * _EW
    lanes = lax.iota(_i32, 16)

    def body(t, carry):
        off = pl.multiple_of(base + t * _CH, _CH)
        pltpu.sync_copy(src_hbm.at[pl.ds(off, _CH)], sv)
        pltpu.sync_copy(dst_hbm.at[pl.ds(off, _CH)], dv)
        cp1 = pltpu.async_copy(mask_hbm.at[sv], smv, sem)
        cp2 = pltpu.async_copy(mask_hbm.at[dv], dmv, sem2)
        cp1.wait()
        cp2.wait()
        for i in range(_CH // 16):
            sl = pl.ds(i * 16, 16)
            keep = (smv[sl] > 0) & (dmv[sl] > 0)
            dump = _N + ((t * 8 + i) % 15) * 16 + lanes
            dpv[sl] = jnp.where(keep, dv[sl], dump)
        pltpu.sync_copy(dpv, out_hbm.at[pl.ds(off, _CH)])
        return carry

    lax.fori_loop(0, _CPW, body, 0)


@functools.lru_cache(maxsize=None)
def _maskedge_kernel():
    return pl.kernel(
        _maskedge_body,
        mesh=_sc_mesh(),
        out_type=jax.ShapeDtypeStruct((_EPAD,), _i32),
        scratch_types=[
            pltpu.VMEM((_CH,), _i32),
            pltpu.VMEM((_CH,), _i32),
            pltpu.VMEM((_CH,), _i32),
            pltpu.VMEM((_CH,), _i32),
            pltpu.VMEM((_CH,), _i32),
            pltpu.SemaphoreType.DMA,
            pltpu.SemaphoreType.DMA,
        ],
    )


@functools.lru_cache(maxsize=None)
def _sagelin(ns, nx, relu, use_mask, use_bias=True):
    def body(*refs):
        it = iter(refs)
        cnt = next(it) if ns else None
        ss = [next(it) for _ in range(ns)]
        wls = [next(it) for _ in range(ns)]
        xs = [next(it) for _ in range(nx)]
        wrs = [next(it) for _ in range(nx)]
        bl = next(it) if use_bias else None
        mk = next(it) if use_mask else None
        out = next(it)
        acc = jnp.zeros((_B, _D), _f32)
        if ns:
            cv = jnp.maximum(cnt[0] + cnt[1], 1.0)
            for sref, wl in zip(ss, wls):
                mean = (sref[0] + sref[1]) / cv
                acc = acc + jnp.dot(mean, wl[...], preferred_element_type=_f32)
        for xref, wr in zip(xs, wrs):
            acc = acc + jnp.dot(xref[...], wr[...], preferred_element_type=_f32)
        if use_bias:
            acc = acc + bl[...]
        if relu:
            acc = jnp.maximum(acc, 0.0)
        if use_mask:
            acc = jnp.where(mk[...] > 0, acc, 0.0)
        out[...] = acc

    in_specs = []
    if ns:
        in_specs.append(pl.BlockSpec((2, _B, 1), lambda i: (0, i, 0)))
        in_specs += [pl.BlockSpec((2, _B, _D), lambda i: (0, i, 0))] * ns
        in_specs += [pl.BlockSpec((_D, _D), lambda i: (0, 0))] * ns
    in_specs += [pl.BlockSpec((_B, _D), lambda i: (i, 0))] * nx
    in_specs += [pl.BlockSpec((_D, _D), lambda i: (0, 0))] * nx
    if use_bias:
        in_specs.append(pl.BlockSpec((1, _D), lambda i: (0, 0)))
    if use_mask:
        in_specs.append(pl.BlockSpec((_B, 1), lambda i: (i, 0)))

    call = pl.pallas_call(
        body,
        grid=(_GRID,),
        in_specs=in_specs,
        out_specs=pl.BlockSpec((_B, _D), lambda i: (i, 0)),
        out_shape=jax.ShapeDtypeStruct((_NP, _D), _f32),
        compiler_params=pltpu.CompilerParams(
            dimension_semantics=("arbitrary",)),
    )

    def run(cnt, ss, wls, xs, wrs, bl, mk):
        args = []
        if ns:
            args.append(cnt.reshape(2, _NP, 1))
            args += list(ss)
            args += list(wls)
        args += list(xs)
        args += list(wrs)
        if use_bias:
            args.append(bl.reshape(1, _D))
        if use_mask:
            args.append(mk)
        return call(*args)

    return run


@functools.lru_cache(maxsize=None)
def _pool(k):
    def body(x_ref, m_ref, p_ref, xg_ref, nm_ref):
        xs = x_ref[...]
        pv = p_ref[...]
        pn = jnp.sqrt(jnp.sum(pv * pv)) + 1e-16
        sc = jnp.dot(xs, pv, preferred_element_type=_f32) / pn
        u = lax.bitcast_convert_type(sc, jnp.uint32)
        key = jnp.where(u >= jnp.uint32(0x80000000), ~u,
                        u | jnp.uint32(0x80000000))
        key = jnp.where(m_ref[...] > 0, key, jnp.uint32(0))
        kf = jnp.float32(k)

        def tb(b, t):
            cand = t | (jnp.uint32(1) << (31 - b).astype(jnp.uint32))
            n_ge = jnp.sum((key >= cand).astype(_f32))
            return jnp.where(n_ge >= kf, cand, t)

        T = lax.fori_loop(0, 32, tb, jnp.uint32(0))
        c_gt = jnp.sum((key > T).astype(_f32))
        need = kf - c_gt
        idx = lax.broadcasted_iota(_i32, (_NP, 1), 0)

        def ib(b, m):
            cand = m + (jnp.int32(1) << (13 - b).astype(_i32))
            f = jnp.sum(((key == T) & (idx <= cand - 1)).astype(_f32))
            return jnp.where(f < need, cand, m)

        m = lax.fori_loop(0, 14, ib, jnp.int32(0))
        newm = (key > T) | ((key == T) & (idx <= m))
        nm_ref[...] = newm.astype(_i32)
        gate = jnp.where(newm, jnp.tanh(sc), 0.0)
        xg_ref[...] = xs * gate

    return pl.pallas_call(
        body,
        out_shape=(jax.ShapeDtypeStruct((_NP, _D), _f32),
                   jax.ShapeDtypeStruct((_NP, 1), _i32)),
    )


def kernel(x, edge_index, batch,
           down0_w1l, down0_b1, down0_w1r, down0_w2l, down0_b2, down0_w2r, pool0_p,
           down1_w1l, down1_b1, down1_w1r, down1_w2l, down1_b2, down1_w2r, pool1_p,
           down2_w1l, down2_b1, down2_w1r, down2_w2l, down2_b2, down2_w2r, pool2_p,
           up0_w1l, up0_b1, up0_w1r, up0_w2l, up0_b2, up0_w2r,
           up1_w1l, up1_b1, up1_w1r, up1_w2l, up1_b2, up1_w2r,
           lin1_w, lin1_b):
    x0p = jnp.concatenate(
        [x + batch[:, None].astype(_f32), jnp.zeros((_NP - _N, _D), _f32)], 0)
    src = edge_index[0]
    dst = edge_index[1]
    srcp = jnp.concatenate(
        [src, jnp.arange(_EPAD - _E, dtype=_i32) % _N])
    dstp0 = jnp.concatenate(
        [dst, _N + (jnp.arange(_EPAD - _E, dtype=_i32) % (_NP - _N))])
    z128 = jnp.zeros((_CH, _D), _f32)
    zrow = jnp.zeros((_RPS,), _f32)
    one128 = jnp.ones((_CH,), _f32)
    m0col = jnp.concatenate(
        [jnp.ones((_N,), _i32), jnp.zeros((_NP - _N,), _i32)]).reshape(_NP, 1)

    def seg(xp, dstp, want_cnt=True):
        return _seg(xp, srcp, dstp, z128, zrow, one128, with_cnt=want_cnt)

    sA, c0 = seg(x0p, dstp0, True)
    h = _sagelin(1, 1, True, False)(c0, [sA], [down0_w1l], [x0p], [down0_w1r],
                                    down0_b1, None)
    s2, _ = seg(h, dstp0)
    x1pre = _sagelin(1, 1, False, True)(c0, [s2], [down0_w2l], [h], [down0_w2r],
                                        down0_b2, m0col)
    xg1, m1col = _pool(5000)(x1pre, m0col, pool0_p.reshape(_D, 1))
    dstp1 = _maskedge_kernel()(m1col.reshape(_NP), srcp, dstp0)

    sB, c1 = seg(xg1, dstp1, True)
    h = _sagelin(1, 1, True, False)(c1, [sB], [down1_w1l], [xg1], [down1_w1r],
                                    down1_b1, None)
    s2, _ = seg(h, dstp1)
    x2pre = _sagelin(1, 1, False, True)(c1, [s2], [down1_w2l], [h], [down1_w2r],
                                        down1_b2, m1col)
    xg2, m2col = _pool(2500)(x2pre, m1col, pool1_p.reshape(_D, 1))
    dstp2 = _maskedge_kernel()(m2col.reshape(_NP), srcp, dstp0)

    sC, c2 = seg(xg2, dstp2, True)
    h = _sagelin(1, 1, True, False)(c2, [sC], [down2_w1l], [xg2], [down2_w1r],
                                    down2_b1, None)
    s2, _ = seg(h, dstp2)
    x3pre = _sagelin(1, 1, False, True)(c2, [s2], [down2_w2l], [h], [down2_w2r],
                                        down2_b2, m2col)
    xg3, _m3 = _pool(1250)(x3pre, m2col, pool2_p.reshape(_D, 1))

    sX, _ = seg(xg3, dstp1)
    h = _sagelin(2, 2, True, False)(
        c1, [sX, sB], [up1_w1l[:_D], up1_w1l[_D:]],
        [xg3, xg1], [up1_w1r[:_D], up1_w1r[_D:]], up1_b1, None)
    s2, _ = seg(h, dstp1)
    xu = _sagelin(1, 1, False, True)(c1, [s2], [up1_w2l], [h], [up1_w2r],
                                     up1_b2, m1col)

    sY, _ = seg(xu, dstp0)
    h = _sagelin(2, 2, True, False)(
        c0, [sY, sA], [up0_w1l[:_D], up0_w1l[_D:]],
        [xu, x0p], [up0_w1r[:_D], up0_w1r[_D:]], up0_b1, None)
    s2, _ = seg(h, dstp0)
    xf = _sagelin(1, 1, False, True)(c0, [s2], [up0_w2l], [h], [up0_w2r],
                                     up0_b2, m0col)

    out = _sagelin(0, 4, True, False)(
        None, [], [], [x0p, xg1, xg2, xf],
        [lin1_w[0:_D], lin1_w[_D:2 * _D], lin1_w[2 * _D:3 * _D], lin1_w[3 * _D:]],
        lin1_b, None)
    return out[:_N]
